# baseline (speedup 1.0000x reference)
import jax
import jax.numpy as jnp
from jax import lax
from jax.experimental import pallas as pl
from jax.experimental.pallas import tpu as pltpu
from jax.experimental.pallas import tpu_sc as plsc

_f32 = jnp.float32

N = 10000
D = 128
DOUT = 40
NC = 2
NS = 16
CHUNK = 128
CH2 = 32
NBUF = 8
RPT = 624
RPT_LAST = N - 15 * RPT
JUNK = N
NJUNK = 32
ACC_ROWS = N + NJUNK
DEG_PAD = 10240
DRT = DEG_PAD // NS


def _sc_degree(dstp2, nch):
    def body(dst_hbm, out_hbm, didx_v, ones_v, zbuf_v, deg_sh):
        cid = lax.axis_index("c")
        sid = lax.axis_index("s")
        wid = cid * NS + sid
        for i in range(DRT // 16):
            zbuf_v[pl.ds(i * 16, 16)] = jnp.zeros((16,), _f32)
        for i in range(CHUNK // 16):
            ones_v[pl.ds(i * 16, 16)] = jnp.ones((16,), _f32)
        pltpu.sync_copy(dst_hbm.at[pl.ds(wid * nch, nch)], didx_v)
        pltpu.sync_copy(zbuf_v, deg_sh.at[pl.ds(sid * DRT, DRT)])
        plsc.subcore_barrier()

        def step(i, c):
            pltpu.sync_copy(ones_v, deg_sh.at[didx_v.at[i]], add=True)
            return c

        lax.fori_loop(0, nch, step, 0)
        plsc.subcore_barrier()
        pltpu.sync_copy(deg_sh.at[pl.ds(sid * DRT, DRT)],
                        out_hbm.at[cid, pl.ds(sid * DRT, DRT)])

    return pl.kernel(
        body,
        out_type=jax.ShapeDtypeStruct((NC, DEG_PAD), _f32),
        mesh=plsc.VectorSubcoreMesh(core_axis_name="c", subcore_axis_name="s"),
        scratch_types=[
            pltpu.VMEM((nch, CHUNK), jnp.int32),
            pltpu.VMEM((CHUNK,), _f32),
            pltpu.VMEM((DRT,), _f32),
            pltpu.VMEM_SHARED((DEG_PAD,), _f32),
        ],
    )(dstp2)


def _sc_scatter(z, srcp2, dstp2, zero_rows, nch):
    def body(z_hbm, src_hbm, dst_hbm, zero_hbm, out_hbm,
             sidx_v, didx_v, rows0, rows1, rows2, rows3,
             rows4, rows5, rows6, rows7, acc_sh,
             g0, g1, g2, g3, g4, g5, g6, g7,
             s0, s1, s2, s3, s4, s5, s6, s7):
        rows = [rows0, rows1, rows2, rows3, rows4, rows5, rows6, rows7]
        gs = [g0, g1, g2, g3, g4, g5, g6, g7]
        ss = [s0, s1, s2, s3, s4, s5, s6, s7]
        cid = lax.axis_index("c")
        sid = lax.axis_index("s")
        wid = cid * NS + sid
        rb = sid * RPT

        @pl.when(sid < NS - 1)
        def _():
            pltpu.sync_copy(zero_hbm.at[pl.ds(0, RPT)],
                            acc_sh.at[pl.ds(rb, RPT)])

        @pl.when(sid == NS - 1)
        def _():
            pltpu.sync_copy(zero_hbm, acc_sh.at[pl.ds(rb, RPT_LAST)])

        plsc.subcore_barrier()

        def gwait(b):
            pltpu.make_async_copy(z_hbm.at[pl.ds(0, CH2)], rows[b],
                                  gs[b]).wait()

        def swait(b):
            pltpu.make_async_copy(z_hbm.at[pl.ds(0, CH2)], rows[b],
                                  ss[b]).wait()

        gr = nch // 8
        ngrp = gr // NBUF
        for h in range(8):
            cb = wid * nch + h * gr
            pltpu.sync_copy(src_hbm.at[pl.ds(cb, gr)], sidx_v)
            pltpu.sync_copy(dst_hbm.at[pl.ds(cb, gr)], didx_v)
            for b in range(NBUF):
                pltpu.async_copy(z_hbm.at[sidx_v.at[b]], rows[b], gs[b])

            def step(jj, c):
                j = jj * NBUF
                for b in range(NBUF):
                    gwait(b)
                    pltpu.async_copy(rows[b], acc_sh.at[didx_v.at[j + b]],
                                     ss[b], add=True)
                for b in range(NBUF):
                    @pl.when(jj < ngrp - 1)
                    def _():
                        swait(b)
                        pltpu.async_copy(z_hbm.at[sidx_v.at[j + NBUF + b]],
                                         rows[b], gs[b])
                return c

            lax.fori_loop(0, ngrp, step, 0)
            for b in range(NBUF):
                swait(b)
        plsc.subcore_barrier()

        @pl.when(sid < NS - 1)
        def _():
            pltpu.sync_copy(acc_sh.at[pl.ds(rb, RPT)],
                            out_hbm.at[cid, pl.ds(rb, RPT)])

        @pl.when(sid == NS - 1)
        def _():
            pltpu.sync_copy(acc_sh.at[pl.ds(rb, RPT_LAST)],
                            out_hbm.at[cid, pl.ds(rb, RPT_LAST)])

    return pl.kernel(
        body,
        out_type=jax.ShapeDtypeStruct((NC, N, D), _f32),
        mesh=plsc.VectorSubcoreMesh(core_axis_name="c", subcore_axis_name="s"),
        scratch_types=[
            pltpu.VMEM((nch // 8, CH2), jnp.int32),
            pltpu.VMEM((nch // 8, CH2), jnp.int32),
            pltpu.VMEM((CH2, D), _f32),
            pltpu.VMEM((CH2, D), _f32),
            pltpu.VMEM((CH2, D), _f32),
            pltpu.VMEM((CH2, D), _f32),
            pltpu.VMEM((CH2, D), _f32),
            pltpu.VMEM((CH2, D), _f32),
            pltpu.VMEM((CH2, D), _f32),
            pltpu.VMEM((CH2, D), _f32),
            pltpu.VMEM_SHARED((ACC_ROWS, D), _f32),
            pltpu.SemaphoreType.DMA,
            pltpu.SemaphoreType.DMA,
            pltpu.SemaphoreType.DMA,
            pltpu.SemaphoreType.DMA,
            pltpu.SemaphoreType.DMA,
            pltpu.SemaphoreType.DMA,
            pltpu.SemaphoreType.DMA,
            pltpu.SemaphoreType.DMA,
            pltpu.SemaphoreType.DMA,
            pltpu.SemaphoreType.DMA,
            pltpu.SemaphoreType.DMA,
            pltpu.SemaphoreType.DMA,
            pltpu.SemaphoreType.DMA,
            pltpu.SemaphoreType.DMA,
            pltpu.SemaphoreType.DMA,
            pltpu.SemaphoreType.DMA,
        ],
    )(z, srcp2, dstp2, zero_rows)


_R = 1000


def _dot(a, b):
    return lax.dot_general(a, b, (((1,), (0,)), ((), ())),
                           precision=lax.Precision.HIGHEST,
                           preferred_element_type=_f32)


def _tc_stage_in(x, W_in, b_in, W1, deg2):
    def body(x_ref, win_ref, bin_ref, w1_ref, deg_ref, z1_ref, dinv_ref):
        h = jnp.maximum(_dot(x_ref[...], win_ref[...]) + bin_ref[...], 0.0)
        dinv = lax.rsqrt(deg_ref[:, 0:1] + deg_ref[:, 1:2] + 1.0)
        z1_ref[...] = _dot(h, w1_ref[...]) * dinv
        dinv_ref[...] = dinv

    return pl.pallas_call(
        body,
        grid=(N // _R,),
        in_specs=[
            pl.BlockSpec((_R, D), lambda i: (i, 0)),
            pl.BlockSpec((D, D), lambda i: (0, 0)),
            pl.BlockSpec((1, D), lambda i: (0, 0)),
            pl.BlockSpec((D, D), lambda i: (0, 0)),
            pl.BlockSpec((_R, 2), lambda i: (i, 0)),
        ],
        out_specs=[
            pl.BlockSpec((_R, D), lambda i: (i, 0)),
            pl.BlockSpec((_R, 1), lambda i: (i, 0)),
        ],
        out_shape=[
            jax.ShapeDtypeStruct((N, D), _f32),
            jax.ShapeDtypeStruct((N, 1), _f32),
        ],
    )(x, W_in, b_in.reshape(1, D), W1, deg2)


def _tc_stage_mid(agg, z, dinv, b, Wn):
    def body(agg_ref, z_ref, dinv_ref, b_ref, w_ref, h_ref, zn_ref):
        dv = dinv_ref[...]
        t = jnp.tanh((agg_ref[0] + agg_ref[1] + z_ref[...]) * dv + b_ref[...])
        h_ref[...] = t
        zn_ref[...] = _dot(t, w_ref[...]) * dv

    return pl.pallas_call(
        body,
        grid=(N // _R,),
        in_specs=[
            pl.BlockSpec((NC, _R, D), lambda i: (0, i, 0)),
            pl.BlockSpec((_R, D), lambda i: (i, 0)),
            pl.BlockSpec((_R, 1), lambda i: (i, 0)),
            pl.BlockSpec((1, D), lambda i: (0, 0)),
            pl.BlockSpec((D, D), lambda i: (0, 0)),
        ],
        out_specs=[
            pl.BlockSpec((_R, D), lambda i: (i, 0)),
            pl.BlockSpec((_R, D), lambda i: (i, 0)),
        ],
        out_shape=[
            jax.ShapeDtypeStruct((N, D), _f32),
            jax.ShapeDtypeStruct((N, D), _f32),
        ],
    )(agg, z, dinv, b.reshape(1, D), Wn)


def _tc_stage_out(agg, z, dinv, b3, W_cls, b_cls):
    def body(agg_ref, z_ref, dinv_ref, b3_ref, wc_ref, bc_ref, h_ref, y_ref):
        t = jnp.tanh((agg_ref[0] + agg_ref[1] + z_ref[...]) * dinv_ref[...]
                     + b3_ref[...])
        h_ref[...] = t
        y_ref[...] = _dot(t, wc_ref[...]) + bc_ref[...]

    return pl.pallas_call(
        body,
        grid=(N // _R,),
        in_specs=[
            pl.BlockSpec((NC, _R, D), lambda i: (0, i, 0)),
            pl.BlockSpec((_R, D), lambda i: (i, 0)),
            pl.BlockSpec((_R, 1), lambda i: (i, 0)),
            pl.BlockSpec((1, D), lambda i: (0, 0)),
            pl.BlockSpec((D, DOUT), lambda i: (0, 0)),
            pl.BlockSpec((1, DOUT), lambda i: (0, 0)),
        ],
        out_specs=[
            pl.BlockSpec((_R, D), lambda i: (i, 0)),
            pl.BlockSpec((_R, DOUT), lambda i: (i, 0)),
        ],
        out_shape=[
            jax.ShapeDtypeStruct((N, D), _f32),
            jax.ShapeDtypeStruct((N, DOUT), _f32),
        ],
    )(agg, z, dinv, b3.reshape(1, D), W_cls, b_cls.reshape(1, DOUT))


def kernel(x, edge_index, W_in, b_in, W1, b1, W2, b2, W3, b3, W_cls, b_cls):
    src = edge_index[0]
    dst = edge_index[1]
    e = src.shape[0]
    grp = NC * NS * CH2 * 16
    e_pad = ((e + grp - 1) // grp) * grp
    nch = e_pad // (NC * NS * CH2)
    nchd = e_pad // (NC * NS * CHUNK)
    pad = e_pad - e
    cyc = jnp.arange(pad, dtype=dst.dtype) % NJUNK
    srcp = jnp.concatenate([src, cyc])
    dstp = jnp.concatenate([dst, JUNK + cyc])
    srcp2 = srcp.reshape(e_pad // CH2, CH2)
    dstp2 = dstp.reshape(e_pad // CH2, CH2)
    dstp2d = dstp.reshape(e_pad // CHUNK, CHUNK)
    zero_rows = jnp.zeros((RPT_LAST, D), _f32)

    deg = _sc_degree(dstp2d, nchd)
    deg2 = deg[:, :N].T

    z1, dinv = _tc_stage_in(x, W_in, b_in, W1, deg2)
    agg1 = _sc_scatter(z1, srcp2, dstp2, zero_rows, nch)
    h1, z2 = _tc_stage_mid(agg1, z1, dinv, b1, W2)
    agg2 = _sc_scatter(z2, srcp2, dstp2, zero_rows, nch)
    h2, z3 = _tc_stage_mid(agg2, z2, dinv, b2, W3)
    agg3 = _sc_scatter(z3, srcp2, dstp2, zero_rows, nch)
    h3, y = _tc_stage_out(agg3, z3, dinv, b3, W_cls, b_cls)
    return (h1, h2, h3, y)


# speedup vs baseline: 1.0506x; 1.0506x over previous
import jax
import jax.numpy as jnp
from jax import lax
from jax.experimental import pallas as pl
from jax.experimental.pallas import tpu as pltpu
from jax.experimental.pallas import tpu_sc as plsc

_f32 = jnp.float32

N = 10000
D = 128
DOUT = 40
NC = 2
NS = 16
CHUNK = 128
CH2 = 64
NBUF = 4
RPT = 624
RPT_LAST = N - 15 * RPT
JUNK = N
NJUNK = 128
ACC_ROWS = N + NJUNK
DEG_PAD = 10240
DRT = DEG_PAD // NS


def _sc_degree(dstp2, nch):
    def body(dst_hbm, out_hbm, didx_v, ones_v, zbuf_v, deg_sh):
        cid = lax.axis_index("c")
        sid = lax.axis_index("s")
        wid = cid * NS + sid
        for i in range(DRT // 16):
            zbuf_v[pl.ds(i * 16, 16)] = jnp.zeros((16,), _f32)
        for i in range(CHUNK // 16):
            ones_v[pl.ds(i * 16, 16)] = jnp.ones((16,), _f32)
        pltpu.sync_copy(dst_hbm.at[pl.ds(wid * nch, nch)], didx_v)
        pltpu.sync_copy(zbuf_v, deg_sh.at[pl.ds(sid * DRT, DRT)])
        plsc.subcore_barrier()

        def step(i, c):
            pltpu.sync_copy(ones_v, deg_sh.at[didx_v.at[i]], add=True)
            return c

        lax.fori_loop(0, nch, step, 0)
        plsc.subcore_barrier()
        pltpu.sync_copy(deg_sh.at[pl.ds(sid * DRT, DRT)],
                        out_hbm.at[cid, pl.ds(sid * DRT, DRT)])

    return pl.kernel(
        body,
        out_type=jax.ShapeDtypeStruct((NC, DEG_PAD), _f32),
        mesh=plsc.VectorSubcoreMesh(core_axis_name="c", subcore_axis_name="s"),
        scratch_types=[
            pltpu.VMEM((nch, CHUNK), jnp.int32),
            pltpu.VMEM((CHUNK,), _f32),
            pltpu.VMEM((DRT,), _f32),
            pltpu.VMEM_SHARED((DEG_PAD,), _f32),
        ],
    )(dstp2)


def _sc_scatter(z, srcp2, dstp2, zero_rows, nch):
    def body(z_hbm, src_hbm, dst_hbm, zero_hbm, out_hbm,
             sidx_v, didx_v, rows0, rows1, rows2, rows3, acc_sh,
             g0, g1, g2, g3, s0, s1, s2, s3):
        rows = [rows0, rows1, rows2, rows3]
        gs = [g0, g1, g2, g3]
        ss = [s0, s1, s2, s3]
        cid = lax.axis_index("c")
        sid = lax.axis_index("s")
        wid = cid * NS + sid
        rb = sid * RPT

        @pl.when(sid < NS - 1)
        def _():
            pltpu.sync_copy(zero_hbm.at[pl.ds(0, RPT)],
                            acc_sh.at[pl.ds(rb, RPT)])

        @pl.when(sid == NS - 1)
        def _():
            pltpu.sync_copy(zero_hbm, acc_sh.at[pl.ds(rb, RPT_LAST)])

        plsc.subcore_barrier()

        def gwait(b):
            pltpu.make_async_copy(z_hbm.at[pl.ds(0, CH2)], rows[b],
                                  gs[b]).wait()

        def swait(b):
            pltpu.make_async_copy(z_hbm.at[pl.ds(0, CH2)], rows[b],
                                  ss[b]).wait()

        gr = nch // 4
        ngrp = gr // NBUF
        for h in range(4):
            cb = wid * nch + h * gr
            pltpu.sync_copy(src_hbm.at[pl.ds(cb, gr)], sidx_v)
            pltpu.sync_copy(dst_hbm.at[pl.ds(cb, gr)], didx_v)
            for b in range(NBUF):
                pltpu.async_copy(z_hbm.at[sidx_v.at[b]], rows[b], gs[b])

            def step(jj, c):
                j = jj * NBUF
                for b in range(NBUF):
                    gwait(b)
                    pltpu.async_copy(rows[b], acc_sh.at[didx_v.at[j + b]],
                                     ss[b], add=True)
                for b in range(NBUF):
                    @pl.when(jj < ngrp - 1)
                    def _():
                        swait(b)
                        pltpu.async_copy(z_hbm.at[sidx_v.at[j + NBUF + b]],
                                         rows[b], gs[b])
                return c

            lax.fori_loop(0, ngrp, step, 0)
            for b in range(NBUF):
                swait(b)
        plsc.subcore_barrier()

        @pl.when(sid < NS - 1)
        def _():
            pltpu.sync_copy(acc_sh.at[pl.ds(rb, RPT)],
                            out_hbm.at[cid, pl.ds(rb, RPT)])

        @pl.when(sid == NS - 1)
        def _():
            pltpu.sync_copy(acc_sh.at[pl.ds(rb, RPT_LAST)],
                            out_hbm.at[cid, pl.ds(rb, RPT_LAST)])

    return pl.kernel(
        body,
        out_type=jax.ShapeDtypeStruct((NC, N, D), _f32),
        mesh=plsc.VectorSubcoreMesh(core_axis_name="c", subcore_axis_name="s"),
        scratch_types=[
            pltpu.VMEM((nch // 4, CH2), jnp.int32),
            pltpu.VMEM((nch // 4, CH2), jnp.int32),
            pltpu.VMEM((CH2, D), _f32),
            pltpu.VMEM((CH2, D), _f32),
            pltpu.VMEM((CH2, D), _f32),
            pltpu.VMEM((CH2, D), _f32),
            pltpu.VMEM_SHARED((ACC_ROWS, D), _f32),
            pltpu.SemaphoreType.DMA,
            pltpu.SemaphoreType.DMA,
            pltpu.SemaphoreType.DMA,
            pltpu.SemaphoreType.DMA,
            pltpu.SemaphoreType.DMA,
            pltpu.SemaphoreType.DMA,
            pltpu.SemaphoreType.DMA,
            pltpu.SemaphoreType.DMA,
        ],
    )(z, srcp2, dstp2, zero_rows)


_R = 1000


def _dot(a, b):
    return lax.dot_general(a, b, (((1,), (0,)), ((), ())),
                           precision=lax.Precision.HIGHEST,
                           preferred_element_type=_f32)


def _tc_stage_in(x, W_in, b_in, W1, deg2):
    def body(x_ref, win_ref, bin_ref, w1_ref, deg_ref, z1_ref, dinv_ref):
        h = jnp.maximum(_dot(x_ref[...], win_ref[...]) + bin_ref[...], 0.0)
        dinv = lax.rsqrt(deg_ref[:, 0:1] + deg_ref[:, 1:2] + 1.0)
        z1_ref[...] = _dot(h, w1_ref[...]) * dinv
        dinv_ref[...] = dinv

    return pl.pallas_call(
        body,
        grid=(N // _R,),
        in_specs=[
            pl.BlockSpec((_R, D), lambda i: (i, 0)),
            pl.BlockSpec((D, D), lambda i: (0, 0)),
            pl.BlockSpec((1, D), lambda i: (0, 0)),
            pl.BlockSpec((D, D), lambda i: (0, 0)),
            pl.BlockSpec((_R, 2), lambda i: (i, 0)),
        ],
        out_specs=[
            pl.BlockSpec((_R, D), lambda i: (i, 0)),
            pl.BlockSpec((_R, 1), lambda i: (i, 0)),
        ],
        out_shape=[
            jax.ShapeDtypeStruct((N, D), _f32),
            jax.ShapeDtypeStruct((N, 1), _f32),
        ],
    )(x, W_in, b_in.reshape(1, D), W1, deg2)


def _tc_stage_mid(agg, z, dinv, b, Wn):
    def body(agg_ref, z_ref, dinv_ref, b_ref, w_ref, h_ref, zn_ref):
        dv = dinv_ref[...]
        t = jnp.tanh((agg_ref[0] + agg_ref[1] + z_ref[...]) * dv + b_ref[...])
        h_ref[...] = t
        zn_ref[...] = _dot(t, w_ref[...]) * dv

    return pl.pallas_call(
        body,
        grid=(N // _R,),
        in_specs=[
            pl.BlockSpec((NC, _R, D), lambda i: (0, i, 0)),
            pl.BlockSpec((_R, D), lambda i: (i, 0)),
            pl.BlockSpec((_R, 1), lambda i: (i, 0)),
            pl.BlockSpec((1, D), lambda i: (0, 0)),
            pl.BlockSpec((D, D), lambda i: (0, 0)),
        ],
        out_specs=[
            pl.BlockSpec((_R, D), lambda i: (i, 0)),
            pl.BlockSpec((_R, D), lambda i: (i, 0)),
        ],
        out_shape=[
            jax.ShapeDtypeStruct((N, D), _f32),
            jax.ShapeDtypeStruct((N, D), _f32),
        ],
    )(agg, z, dinv, b.reshape(1, D), Wn)


def _tc_stage_out(agg, z, dinv, b3, W_cls, b_cls):
    def body(agg_ref, z_ref, dinv_ref, b3_ref, wc_ref, bc_ref, h_ref, y_ref):
        t = jnp.tanh((agg_ref[0] + agg_ref[1] + z_ref[...]) * dinv_ref[...]
                     + b3_ref[...])
        h_ref[...] = t
        y_ref[...] = _dot(t, wc_ref[...]) + bc_ref[...]

    return pl.pallas_call(
        body,
        grid=(N // _R,),
        in_specs=[
            pl.BlockSpec((NC, _R, D), lambda i: (0, i, 0)),
            pl.BlockSpec((_R, D), lambda i: (i, 0)),
            pl.BlockSpec((_R, 1), lambda i: (i, 0)),
            pl.BlockSpec((1, D), lambda i: (0, 0)),
            pl.BlockSpec((D, DOUT), lambda i: (0, 0)),
            pl.BlockSpec((1, DOUT), lambda i: (0, 0)),
        ],
        out_specs=[
            pl.BlockSpec((_R, D), lambda i: (i, 0)),
            pl.BlockSpec((_R, DOUT), lambda i: (i, 0)),
        ],
        out_shape=[
            jax.ShapeDtypeStruct((N, D), _f32),
            jax.ShapeDtypeStruct((N, DOUT), _f32),
        ],
    )(agg, z, dinv, b3.reshape(1, D), W_cls, b_cls.reshape(1, DOUT))


def kernel(x, edge_index, W_in, b_in, W1, b1, W2, b2, W3, b3, W_cls, b_cls):
    src = edge_index[0]
    dst = edge_index[1]
    e = src.shape[0]
    grp = NC * NS * CH2 * 16
    e_pad = ((e + grp - 1) // grp) * grp
    nch = e_pad // (NC * NS * CH2)
    nchd = e_pad // (NC * NS * CHUNK)
    pad = e_pad - e
    cyc = jnp.arange(pad, dtype=dst.dtype) % NJUNK
    srcp = jnp.concatenate([src, cyc])
    dstp = jnp.concatenate([dst, JUNK + cyc])
    srcp2 = srcp.reshape(e_pad // CH2, CH2)
    dstp2 = dstp.reshape(e_pad // CH2, CH2)
    dstp2d = dstp.reshape(e_pad // CHUNK, CHUNK)
    zero_rows = jnp.zeros((RPT_LAST, D), _f32)

    deg = _sc_degree(dstp2d, nchd)
    deg2 = deg[:, :N].T

    z1, dinv = _tc_stage_in(x, W_in, b_in, W1, deg2)
    agg1 = _sc_scatter(z1, srcp2, dstp2, zero_rows, nch)
    h1, z2 = _tc_stage_mid(agg1, z1, dinv, b1, W2)
    agg2 = _sc_scatter(z2, srcp2, dstp2, zero_rows, nch)
    h2, z3 = _tc_stage_mid(agg2, z2, dinv, b2, W3)
    agg3 = _sc_scatter(z3, srcp2, dstp2, zero_rows, nch)
    h3, y = _tc_stage_out(agg3, z3, dinv, b3, W_cls, b_cls)
    return (h1, h2, h3, y)
